# Initial kernel scaffold; baseline (speedup 1.0000x reference)
import jax
import jax.numpy as jnp
from jax.experimental import pallas as pl


def kernel(batch_h, batch_t, batch_r, tail_emb, rel_emb):
    raise NotImplementedError("write your pallas kernel here")


# speedup vs baseline: 1.8054x; 1.8054x over previous
import functools

import jax
import jax.numpy as jnp
from jax import lax
from jax.experimental import pallas as pl
from jax.experimental.pallas import tpu as pltpu
from jax.experimental.pallas import tpu_sc as plsc

B = 16384
D = 128
L = 16
NC = 2
NS = 16
NW = NC * NS
BPW = B // NW
C = 128
NCHUNK = BPW // C


def _rsqrt(s):
    i = plsc.bitcast(s, jnp.int32)
    i = jnp.int32(0x5F3759DF) - (i >> 1)
    y = plsc.bitcast(i, jnp.float32)
    for _ in range(3):
        y = y * (1.5 - 0.5 * s * y * y)
    return y


def _hsum(x, lane):
    for sh in (8, 4, 2, 1):
        x = x + x.at[lane ^ sh].get(mode="promise_in_bounds")
    return x


def _sc_kernel(h_hbm, t_hbm, r_hbm, tail_hbm, rel_hbm, out_hbm,
               hi_v, ti_v, ri_v, hrows, trows, rrows, outc, sem):
    wid = lax.axis_index("s") * NC + lax.axis_index("c")
    base = wid * BPW
    lane = lax.iota(jnp.int32, L)
    lane0 = lane == 0

    for k in range(NCHUNK):
        off = base + k * C
        pltpu.sync_copy(h_hbm.at[pl.ds(off, C)], hi_v)
        pltpu.sync_copy(t_hbm.at[pl.ds(off, C)], ti_v)
        pltpu.sync_copy(r_hbm.at[pl.ds(off, C)], ri_v)
        cp_h = pltpu.async_copy(tail_hbm.at[hi_v], hrows, sem)
        cp_t = pltpu.async_copy(tail_hbm.at[ti_v], trows, sem)
        cp_r = pltpu.async_copy(rel_hbm.at[ri_v], rrows, sem)
        cp_h.wait()
        cp_t.wait()
        cp_r.wait()

        def row_body(i, carry):
            hv = [hrows[i, pl.ds(j * L, L)] for j in range(D // L)]
            tv = [trows[i, pl.ds(j * L, L)] for j in range(D // L)]
            rv = [rrows[i, pl.ds(j * L, L)] for j in range(D // L)]
            sh = hv[0] * hv[0]
            st = tv[0] * tv[0]
            sr = rv[0] * rv[0]
            for j in range(1, D // L):
                sh = sh + hv[j] * hv[j]
                st = st + tv[j] * tv[j]
                sr = sr + rv[j] * rv[j]
            ih = _rsqrt(jnp.maximum(_hsum(sh, lane), 1e-24))
            it = _rsqrt(jnp.maximum(_hsum(st, lane), 1e-24))
            ir = _rsqrt(jnp.maximum(_hsum(sr, lane), 1e-24))
            acc = jnp.abs(hv[0] * ih + rv[0] * ir - tv[0] * it)
            for j in range(1, D // L):
                acc = acc + jnp.abs(hv[j] * ih + rv[j] * ir - tv[j] * it)
            score = _hsum(acc, lane)
            plsc.store_scatter(outc, [jnp.full((L,), i, jnp.int32)],
                               score, mask=lane0)
            return carry

        lax.fori_loop(0, C, row_body, 0, unroll=False)
        pltpu.sync_copy(outc, out_hbm.at[pl.ds(off, C)])


def kernel(batch_h, batch_t, batch_r, tail_emb, rel_emb):
    mesh = plsc.VectorSubcoreMesh(core_axis_name="c", subcore_axis_name="s")
    f = functools.partial(
        pl.kernel,
        mesh=mesh,
        out_type=jax.ShapeDtypeStruct((B,), jnp.float32),
        compiler_params=pltpu.CompilerParams(needs_layout_passes=False),
        scratch_types=[
            pltpu.VMEM((C,), jnp.int32),
            pltpu.VMEM((C,), jnp.int32),
            pltpu.VMEM((C,), jnp.int32),
            pltpu.VMEM((C, D), jnp.float32),
            pltpu.VMEM((C, D), jnp.float32),
            pltpu.VMEM((C, D), jnp.float32),
            pltpu.VMEM((C,), jnp.float32),
            pltpu.SemaphoreType.DMA,
        ],
    )(_sc_kernel)
    return f(batch_h.astype(jnp.int32), batch_t.astype(jnp.int32),
             batch_r.astype(jnp.int32), tail_emb, rel_emb)


# speedup vs baseline: 1.8571x; 1.0286x over previous
import functools

import jax
import jax.numpy as jnp
from jax import lax
from jax.experimental import pallas as pl
from jax.experimental.pallas import tpu as pltpu
from jax.experimental.pallas import tpu_sc as plsc

B = 16384
D = 128
L = 16
NV = D // L
NC = 2
NS = 16
NW = NC * NS
BPW = B // NW
C = 128
NCHUNK = BPW // C
NG = C // L


def _rsqrt(s):
    i = plsc.bitcast(s, jnp.int32)
    i = jnp.int32(0x5F3759DF) - (i >> 1)
    y = plsc.bitcast(i, jnp.float32)
    for _ in range(3):
        y = y * (1.5 - 0.5 * s * y * y)
    return y


def _perm(x, idx):
    return x.at[idx].get(mode="promise_in_bounds")


def _merge(a, b, m, pk):
    return jnp.where(m, _perm(b, pk), a) + jnp.where(m, b, _perm(a, pk))


def _reduce16(vecs, lane):
    for k in (1, 2, 4, 8):
        m = (lane & k) != 0
        pk = lane ^ k
        vecs = [_merge(vecs[2 * i], vecs[2 * i + 1], m, pk)
                for i in range(len(vecs) // 2)]
    return vecs[0]


def _sc_kernel(h_hbm, t_hbm, r_hbm, tail_hbm, rel_hbm, out_hbm,
               hi_v, ti_v, ri_v, rows, pb, ib, sb, outc, sems):
    wid = lax.axis_index("s") * NC + lax.axis_index("c")
    lane = lax.iota(jnp.int32, L)

    pltpu.sync_copy(h_hbm.at[wid], hi_v)
    pltpu.sync_copy(t_hbm.at[wid], ti_v)
    pltpu.sync_copy(r_hbm.at[wid], ri_v)

    def start_gathers(k, slot):
        cps = [
            pltpu.async_copy(tail_hbm.at[hi_v.at[k]], rows.at[slot, 0], sems.at[slot]),
            pltpu.async_copy(tail_hbm.at[ti_v.at[k]], rows.at[slot, 1], sems.at[slot]),
            pltpu.async_copy(rel_hbm.at[ri_v.at[k]], rows.at[slot, 2], sems.at[slot]),
        ]
        return cps

    pend = start_gathers(0, 0)

    for k in range(NCHUNK):
        slot = k % 2
        for cp in pend:
            cp.wait()
        if k + 1 < NCHUNK:
            pend = start_gathers(k + 1, 1 - slot)

        hrows = rows.at[slot, 0]
        trows = rows.at[slot, 1]
        rrows = rows.at[slot, 2]

        def p1_body(i, carry):
            for x, rbuf in ((hrows, 0), (trows, 1), (rrows, 2)):
                v = x[i, pl.ds(0, L)]
                s = v * v
                for j in range(1, NV):
                    v = x[i, pl.ds(j * L, L)]
                    s = s + v * v
                pb[rbuf, i] = s
            return carry

        lax.fori_loop(0, C, p1_body, 0, unroll=2)

        def p2_body(g, carry):
            for rbuf in range(3):
                vecs = [pb[rbuf, g * L + m] for m in range(L)]
                ssq = _reduce16(vecs, lane)
                ib[rbuf, g] = _rsqrt(jnp.maximum(ssq, 1e-24))
            return carry

        lax.fori_loop(0, NG, p2_body, 0, unroll=False)

        def p3_body(g, carry):
            ihv = ib[0, g]
            itv = ib[1, g]
            irv = ib[2, g]
            for m in range(L):
                i = g * L + m
                splat_m = jnp.full((L,), m, jnp.int32)
                ih = _perm(ihv, splat_m)
                it = _perm(itv, splat_m)
                ir = _perm(irv, splat_m)
                acc = jnp.abs(hrows[i, pl.ds(0, L)] * ih
                              + rrows[i, pl.ds(0, L)] * ir
                              - trows[i, pl.ds(0, L)] * it)
                for j in range(1, NV):
                    acc = acc + jnp.abs(hrows[i, pl.ds(j * L, L)] * ih
                                        + rrows[i, pl.ds(j * L, L)] * ir
                                        - trows[i, pl.ds(j * L, L)] * it)
                sb[i] = acc
            return carry

        lax.fori_loop(0, NG, p3_body, 0, unroll=False)

        def p4_body(g, carry):
            vecs = [sb[g * L + m] for m in range(L)]
            outc[pl.ds(g * L, L)] = _reduce16(vecs, lane)
            return carry

        lax.fori_loop(0, NG, p4_body, 0, unroll=False)

        pltpu.sync_copy(outc, out_hbm.at[wid, pl.ds(k * C, C)])


def kernel(batch_h, batch_t, batch_r, tail_emb, rel_emb):
    mesh = plsc.VectorSubcoreMesh(core_axis_name="c", subcore_axis_name="s")
    f = functools.partial(
        pl.kernel,
        mesh=mesh,
        out_type=jax.ShapeDtypeStruct((NW, BPW), jnp.float32),
        compiler_params=pltpu.CompilerParams(needs_layout_passes=False,
                                             use_tc_tiling_on_sc=False),
        scratch_types=[
            pltpu.VMEM((NCHUNK, C), jnp.int32),
            pltpu.VMEM((NCHUNK, C), jnp.int32),
            pltpu.VMEM((NCHUNK, C), jnp.int32),
            pltpu.VMEM((2, 3, C, D), jnp.float32),
            pltpu.VMEM((3, C, L), jnp.float32),
            pltpu.VMEM((3, NG, L), jnp.float32),
            pltpu.VMEM((C, L), jnp.float32),
            pltpu.VMEM((C,), jnp.float32),
            pltpu.SemaphoreType.DMA((2,)),
        ],
    )(_sc_kernel)
    out = f(batch_h.astype(jnp.int32).reshape(NW, NCHUNK, C),
            batch_t.astype(jnp.int32).reshape(NW, NCHUNK, C),
            batch_r.astype(jnp.int32).reshape(NW, NCHUNK, C),
            tail_emb, rel_emb)
    return out.reshape(B)


# speedup vs baseline: 2.1439x; 1.1545x over previous
import functools

import jax
import jax.numpy as jnp
from jax import lax
from jax.experimental import pallas as pl
from jax.experimental.pallas import tpu as pltpu
from jax.experimental.pallas import tpu_sc as plsc

B = 16384
D = 128
L = 16
NV = D // L
NC = 2
NS = 16
NW = NC * NS
BPW = B // NW
C = 128
NCHUNK = BPW // C
NG = C // L


def _rsqrt(s):
    i = plsc.bitcast(s, jnp.int32)
    i = jnp.int32(0x5F3759DF) - (i >> 1)
    y = plsc.bitcast(i, jnp.float32)
    for _ in range(3):
        y = y * (1.5 - 0.5 * s * y * y)
    return y


def _perm(x, idx):
    return x.at[idx].get(mode="promise_in_bounds")


def _merge(a, b, m, pk):
    return jnp.where(m, _perm(b, pk), a) + jnp.where(m, b, _perm(a, pk))


def _reduce16(vecs, lane):
    for k in (1, 2, 4, 8):
        m = (lane & k) != 0
        pk = lane ^ k
        vecs = [_merge(vecs[2 * i], vecs[2 * i + 1], m, pk)
                for i in range(len(vecs) // 2)]
    return vecs[0]


def _sc_kernel(h_hbm, t_hbm, r_hbm, tail_hbm, rel_hbm, out_hbm,
               hi_v, ti_v, ri_v, rows, pb, ib, sb, outc, sems):
    wid = lax.axis_index("s") * NC + lax.axis_index("c")
    base = wid * BPW
    lane = lax.iota(jnp.int32, L)

    pltpu.sync_copy(h_hbm.at[pl.ds(base, BPW)], hi_v)
    pltpu.sync_copy(t_hbm.at[pl.ds(base, BPW)], ti_v)
    pltpu.sync_copy(r_hbm.at[pl.ds(base, BPW)], ri_v)

    def start_gathers(k, slot):
        return [
            pltpu.async_copy(tail_hbm.at[hi_v.at[pl.ds(k * C, C)]],
                             rows.at[slot, 0], sems.at[slot]),
            pltpu.async_copy(tail_hbm.at[ti_v.at[pl.ds(k * C, C)]],
                             rows.at[slot, 1], sems.at[slot]),
            pltpu.async_copy(rel_hbm.at[ri_v.at[pl.ds(k * C, C)]],
                             rows.at[slot, 2], sems.at[slot]),
        ]

    pend = start_gathers(0, 0)

    for k in range(NCHUNK):
        slot = k % 2
        for cp in pend:
            cp.wait()
        if k + 1 < NCHUNK:
            pend = start_gathers(k + 1, 1 - slot)

        hrows = rows.at[slot, 0]
        trows = rows.at[slot, 1]
        rrows = rows.at[slot, 2]

        def p1_body(i, carry):
            acc = [[None, None], [None, None], [None, None]]
            for j in range(NV):
                vs = [hrows[i, pl.ds(j * L, L)],
                      trows[i, pl.ds(j * L, L)],
                      rrows[i, pl.ds(j * L, L)]]
                for x in range(3):
                    q = vs[x] * vs[x]
                    a = acc[x][j & 1]
                    acc[x][j & 1] = q if a is None else a + q
            pb[0, i] = acc[0][0] + acc[0][1]
            pb[1, i] = acc[1][0] + acc[1][1]
            pb[2, i] = acc[2][0] + acc[2][1]
            return carry

        lax.fori_loop(0, C, p1_body, 0, unroll=2)

        def p2_body(g, carry):
            for rbuf in range(3):
                vecs = [pb[rbuf, g * L + m] for m in range(L)]
                ssq = _reduce16(vecs, lane)
                ib[rbuf, g] = _rsqrt(jnp.maximum(ssq, 1e-24))
            return carry

        lax.fori_loop(0, NG, p2_body, 0, unroll=False)

        def p3_body(g, carry):
            ihv = ib[0, g]
            itv = ib[1, g]
            irv = ib[2, g]
            for m in range(L):
                i = g * L + m
                splat_m = jnp.full((L,), m, jnp.int32)
                ih = _perm(ihv, splat_m)
                it = _perm(itv, splat_m)
                ir = _perm(irv, splat_m)
                terms = [jnp.abs(hrows[i, pl.ds(j * L, L)] * ih
                                 + rrows[i, pl.ds(j * L, L)] * ir
                                 - trows[i, pl.ds(j * L, L)] * it)
                         for j in range(NV)]
                while len(terms) > 1:
                    terms = [terms[2 * n] + terms[2 * n + 1]
                             for n in range(len(terms) // 2)]
                sb[i] = terms[0]
            return carry

        lax.fori_loop(0, NG, p3_body, 0, unroll=False)

        def p4_body(g, carry):
            vecs = [sb[g * L + m] for m in range(L)]
            outc[pl.ds(g * L, L)] = _reduce16(vecs, lane)
            return carry

        lax.fori_loop(0, NG, p4_body, 0, unroll=False)

        pltpu.sync_copy(outc, out_hbm.at[pl.ds(base + k * C, C)])


def kernel(batch_h, batch_t, batch_r, tail_emb, rel_emb):
    mesh = plsc.VectorSubcoreMesh(core_axis_name="c", subcore_axis_name="s")
    f = functools.partial(
        pl.kernel,
        mesh=mesh,
        out_type=jax.ShapeDtypeStruct((B,), jnp.float32),
        compiler_params=pltpu.CompilerParams(needs_layout_passes=False,
                                             use_tc_tiling_on_sc=False),
        scratch_types=[
            pltpu.VMEM((BPW,), jnp.int32),
            pltpu.VMEM((BPW,), jnp.int32),
            pltpu.VMEM((BPW,), jnp.int32),
            pltpu.VMEM((2, 3, C, D), jnp.float32),
            pltpu.VMEM((3, C, L), jnp.float32),
            pltpu.VMEM((3, NG, L), jnp.float32),
            pltpu.VMEM((C, L), jnp.float32),
            pltpu.VMEM((C,), jnp.float32),
            pltpu.SemaphoreType.DMA((2,)),
        ],
    )(_sc_kernel)
    return f(batch_h.astype(jnp.int32), batch_t.astype(jnp.int32),
             batch_r.astype(jnp.int32), tail_emb, rel_emb)


# speedup vs baseline: 2.3065x; 1.0758x over previous
import functools

import jax
import jax.numpy as jnp
from jax import lax
from jax.experimental import pallas as pl
from jax.experimental.pallas import tpu as pltpu
from jax.experimental.pallas import tpu_sc as plsc

B = 16384
D = 128
L = 16
NV = D // L
NC = 2
NS = 16
NW = NC * NS
BPW = B // NW
C = 128
NCHUNK = BPW // C
NG = C // L


def _rsqrt(s):
    i = plsc.bitcast(s, jnp.int32)
    i = jnp.int32(0x5F3759DF) - (i >> 1)
    y = plsc.bitcast(i, jnp.float32)
    for _ in range(3):
        y = y * (1.5 - 0.5 * s * y * y)
    return y


def _perm(x, idx):
    return x.at[idx].get(mode="promise_in_bounds")


def _merge(a, b, m, pk):
    return jnp.where(m, _perm(b, pk), a) + jnp.where(m, b, _perm(a, pk))


def _reduce16(vecs, lane):
    for k in (1, 2, 4, 8):
        m = (lane & k) != 0
        pk = lane ^ k
        vecs = [_merge(vecs[2 * i], vecs[2 * i + 1], m, pk)
                for i in range(len(vecs) // 2)]
    return vecs[0]


def _sc_kernel(h_hbm, t_hbm, r_hbm, tail_hbm, rel_hbm, out_hbm,
               hi_v, ti_v, ri_v, rows, pb, ib, sb, outc, sems):
    wid = lax.axis_index("s") * NC + lax.axis_index("c")
    base = wid * BPW
    lane = lax.iota(jnp.int32, L)

    pltpu.sync_copy(h_hbm.at[pl.ds(base, BPW)], hi_v)
    pltpu.sync_copy(t_hbm.at[pl.ds(base, BPW)], ti_v)
    pltpu.sync_copy(r_hbm.at[pl.ds(base, BPW)], ri_v)

    def make_gathers(k, slot):
        return [
            pltpu.make_async_copy(tail_hbm.at[hi_v.at[pl.ds(k * C, C)]],
                                  rows.at[slot, 0], sems.at[slot]),
            pltpu.make_async_copy(tail_hbm.at[ti_v.at[pl.ds(k * C, C)]],
                                  rows.at[slot, 1], sems.at[slot]),
            pltpu.make_async_copy(rel_hbm.at[ri_v.at[pl.ds(k * C, C)]],
                                  rows.at[slot, 2], sems.at[slot]),
        ]

    for cp in make_gathers(0, 0):
        cp.start()

    def chunk_pair(kk, carry):
        for b in range(2):
            k = kk * 2 + b
            slot = b
            for cp in make_gathers(k, slot):
                cp.wait()
            if b == 0:
                for cp in make_gathers(k + 1, 1 - slot):
                    cp.start()
            else:
                @pl.when(kk != NCHUNK // 2 - 1)
                def _():
                    for cp in make_gathers(k + 1, 1 - slot):
                        cp.start()
            _compute_chunk(k, slot)
        return carry

    def _compute_chunk(k, slot):
        hrows = rows.at[slot, 0]
        trows = rows.at[slot, 1]
        rrows = rows.at[slot, 2]

        def p1_body(i, carry):
            acc = [[None, None], [None, None], [None, None]]
            for j in range(NV):
                vs = [hrows[i, pl.ds(j * L, L)],
                      trows[i, pl.ds(j * L, L)],
                      rrows[i, pl.ds(j * L, L)]]
                for x in range(3):
                    q = vs[x] * vs[x]
                    a = acc[x][j & 1]
                    acc[x][j & 1] = q if a is None else a + q
            pb[0, i] = acc[0][0] + acc[0][1]
            pb[1, i] = acc[1][0] + acc[1][1]
            pb[2, i] = acc[2][0] + acc[2][1]
            return carry

        lax.fori_loop(0, C, p1_body, 0, unroll=2)

        def p2_body(g, carry):
            for rbuf in range(3):
                vecs = [pb[rbuf, g * L + m] for m in range(L)]
                ssq = _reduce16(vecs, lane)
                ib[rbuf, g] = _rsqrt(jnp.maximum(ssq, 1e-24))
            return carry

        lax.fori_loop(0, NG, p2_body, 0, unroll=False)

        def p3_body(g, carry):
            ihv = ib[0, g]
            itv = ib[1, g]
            irv = ib[2, g]
            for m in range(L):
                i = g * L + m
                splat_m = jnp.full((L,), m, jnp.int32)
                ih = _perm(ihv, splat_m)
                it = _perm(itv, splat_m)
                ir = _perm(irv, splat_m)
                terms = [jnp.abs(hrows[i, pl.ds(j * L, L)] * ih
                                 + rrows[i, pl.ds(j * L, L)] * ir
                                 - trows[i, pl.ds(j * L, L)] * it)
                         for j in range(NV)]
                while len(terms) > 1:
                    terms = [terms[2 * n] + terms[2 * n + 1]
                             for n in range(len(terms) // 2)]
                sb[i] = terms[0]
            return carry

        lax.fori_loop(0, NG, p3_body, 0, unroll=False)

        def p4_body(g, carry):
            vecs = [sb[g * L + m] for m in range(L)]
            outc[pl.ds(g * L, L)] = _reduce16(vecs, lane)
            return carry

        lax.fori_loop(0, NG, p4_body, 0, unroll=False)

        pltpu.sync_copy(outc, out_hbm.at[pl.ds(base + k * C, C)])

    lax.fori_loop(0, NCHUNK // 2, chunk_pair, 0, unroll=False)


def kernel(batch_h, batch_t, batch_r, tail_emb, rel_emb):
    mesh = plsc.VectorSubcoreMesh(core_axis_name="c", subcore_axis_name="s")
    f = functools.partial(
        pl.kernel,
        mesh=mesh,
        out_type=jax.ShapeDtypeStruct((B,), jnp.float32),
        compiler_params=pltpu.CompilerParams(needs_layout_passes=False,
                                             use_tc_tiling_on_sc=False),
        scratch_types=[
            pltpu.VMEM((BPW,), jnp.int32),
            pltpu.VMEM((BPW,), jnp.int32),
            pltpu.VMEM((BPW,), jnp.int32),
            pltpu.VMEM((2, 3, C, D), jnp.float32),
            pltpu.VMEM((3, C, L), jnp.float32),
            pltpu.VMEM((3, NG, L), jnp.float32),
            pltpu.VMEM((C, L), jnp.float32),
            pltpu.VMEM((C,), jnp.float32),
            pltpu.SemaphoreType.DMA((2,)),
        ],
    )(_sc_kernel)
    return f(batch_h.astype(jnp.int32), batch_t.astype(jnp.int32),
             batch_r.astype(jnp.int32), tail_emb, rel_emb)
